# baseline (speedup 1.0000x reference)
import functools

import jax
import jax.numpy as jnp
from jax import lax
from jax.experimental import pallas as pl


def _bias_body(ts_ref, ch_ref, pw_ref, cw_ref, o_ref):
    n_tok = jnp.sum(ts_ref[...])
    n_ch = jnp.sum(ch_ref[...])
    max_n, emb = pw_ref.shape
    max_c = cw_ref.shape[0]
    rows_p = lax.broadcasted_iota(jnp.int32, (max_n, emb), 0)
    sel_p = jnp.where(rows_p < n_tok, pw_ref[...], pw_ref[max_n - 1:max_n, :])
    rows_c = lax.broadcasted_iota(jnp.int32, (max_c, emb), 0)
    sel_c = jnp.where(rows_c < n_ch, cw_ref[...], cw_ref[max_c - 1:max_c, :])
    period = max_c * max_n
    reps = o_ref.shape[0] // period
    for r in range(reps):
        for c in range(max_c):
            base = r * period + c * max_n
            o_ref[base:base + max_n, :] = sel_p + sel_c[c:c + 1, :]


def _add_body(b_ref, x_ref, o_ref):
    o_ref[...] = x_ref[...] + b_ref[...]


@functools.partial(jax.jit, static_argnames=("reps",))
def _run(x, ts_i, ch_i, patch_pos_w, ch_pos_w, reps=8):
    bs, max_c, max_n, emb = x.shape
    period = max_c * max_n
    blk = period * reps
    rows = bs * period
    bias = pl.pallas_call(
        _bias_body,
        out_shape=jax.ShapeDtypeStruct((blk, emb), x.dtype),
    )(ts_i, ch_i, patch_pos_w, ch_pos_w)
    xr = x.reshape(rows, emb)
    out = pl.pallas_call(
        _add_body,
        grid=(rows // blk,),
        in_specs=[
            pl.BlockSpec((blk, emb), lambda i: (0, 0)),
            pl.BlockSpec((blk, emb), lambda i: (i, 0)),
        ],
        out_specs=pl.BlockSpec((blk, emb), lambda i: (i, 0)),
        out_shape=jax.ShapeDtypeStruct((rows, emb), x.dtype),
    )(bias, xr)
    return out.reshape(bs, max_c, max_n, emb)


def kernel(x, ts_token_mask, ch_mask, patch_pos_w, ch_pos_w):
    ts_i = ts_token_mask.astype(jnp.int32)
    ch_i = ch_mask.astype(jnp.int32)
    return _run(x, ts_i, ch_i, patch_pos_w, ch_pos_w)


# speedup vs baseline: 2.2475x; 2.2475x over previous
import functools

import jax
import jax.numpy as jnp
from jax import lax
from jax.experimental import pallas as pl
from jax.experimental.pallas import tpu as pltpu


def _bias_body(ts_ref, ch_ref, pw_ref, cw_ref, o_ref):
    n_tok = jnp.sum(ts_ref[...])
    n_ch = jnp.sum(ch_ref[...])
    max_n, emb = pw_ref.shape
    max_c = cw_ref.shape[0]
    rows_p = lax.broadcasted_iota(jnp.int32, (max_n, emb), 0)
    sel_p = jnp.where(rows_p < n_tok, pw_ref[...], pw_ref[max_n - 1:max_n, :])
    rows_c = lax.broadcasted_iota(jnp.int32, (max_c, emb), 0)
    sel_c = jnp.where(rows_c < n_ch, cw_ref[...], cw_ref[max_c - 1:max_c, :])
    o_ref[...] = sel_c[:, None, :] + sel_p[None, :, :]


def _make_stream_body(bs, bb, ring):
    nb = bs // bb

    def _body(b_ref, x_hbm, o_hbm, xbuf, obuf, in_sems, out_sems):
        def in_copy(i, k):
            return pltpu.make_async_copy(
                x_hbm.at[pl.ds(i * bb, bb)], xbuf.at[k], in_sems.at[k])

        def out_copy(i, k):
            return pltpu.make_async_copy(
                obuf.at[k], o_hbm.at[pl.ds(i * bb, bb)], out_sems.at[k])

        bias = b_ref[...][None]
        for i in range(min(ring, nb)):
            in_copy(i, i % ring).start()
        for i in range(nb):
            k = i % ring
            in_copy(i, k).wait()
            if i >= ring:
                out_copy(i - ring, k).wait()
            obuf[k] = xbuf[k] + bias
            out_copy(i, k).start()
            if i + ring < nb:
                in_copy(i + ring, k).start()
        for i in range(max(nb - ring, 0), nb):
            out_copy(i, i % ring).wait()

    return _body


@functools.partial(jax.jit, static_argnames=("bb", "ring"))
def _run(x, ts_i, ch_i, patch_pos_w, ch_pos_w, bb=16, ring=8):
    bs, max_c, max_n, emb = x.shape
    bias = pl.pallas_call(
        _bias_body,
        out_shape=jax.ShapeDtypeStruct((max_c, max_n, emb), x.dtype),
    )(ts_i, ch_i, patch_pos_w, ch_pos_w)
    out = pl.pallas_call(
        _make_stream_body(bs, bb, ring),
        in_specs=[
            pl.BlockSpec(memory_space=pltpu.VMEM),
            pl.BlockSpec(memory_space=pl.ANY),
        ],
        out_specs=pl.BlockSpec(memory_space=pl.ANY),
        out_shape=jax.ShapeDtypeStruct((bs, max_c, max_n, emb), x.dtype),
        scratch_shapes=[
            pltpu.VMEM((ring, bb, max_c, max_n, emb), x.dtype),
            pltpu.VMEM((ring, bb, max_c, max_n, emb), x.dtype),
            pltpu.SemaphoreType.DMA((ring,)),
            pltpu.SemaphoreType.DMA((ring,)),
        ],
    )(bias, x)
    return out


def kernel(x, ts_token_mask, ch_mask, patch_pos_w, ch_pos_w):
    ts_i = ts_token_mask.astype(jnp.int32)
    ch_i = ch_mask.astype(jnp.int32)
    return _run(x, ts_i, ch_i, patch_pos_w, ch_pos_w)
